# Initial kernel scaffold; baseline (speedup 1.0000x reference)
import jax
import jax.numpy as jnp
from jax.experimental import pallas as pl


def kernel(x, keys, usage, s):
    raise NotImplementedError("write your pallas kernel here")


# speedup vs baseline: 4.1112x; 4.1112x over previous
import jax
import jax.numpy as jnp
from jax.experimental import pallas as pl
from jax.experimental.pallas import tpu as pltpu

_D = 512
_M = 32768
_B = 1024
_TOPK = 8
_GAMMA = 0.1
_THETA = 0.5
_BETA = 10.0

_BLK = 1024
_NBLK = _M // _BLK
_NEG = float("-inf")


def _fused_kernel(xn_ref, keys_ref, usage_ref, s_ref,
                  inh_ref, win_ref, ms_ref, y_ref, g_ref,
                  vals_s, idx_s, u_s):
    phase = pl.program_id(0)
    j = pl.program_id(1)

    cols = jax.lax.broadcasted_iota(jnp.int32, (_B, _BLK), 1) + j * _BLK
    bias = (-_GAMMA) * usage_ref[0, :] + jnp.log(s_ref[0, :] + 1e-6)

    @pl.when(phase == 0)
    def _compute():
        scores = jax.lax.dot_general(
            xn_ref[...], keys_ref[...],
            dimension_numbers=(((1,), (1,)), ((), ())),
            preferred_element_type=jnp.float32)
        biased = scores + bias[None, :]

        work = biased
        bvals = []
        bidx = []
        for _ in range(_TOPK):
            m = jnp.max(work, axis=1, keepdims=True)
            idx = jnp.min(jnp.where(work == m, cols, _M), axis=1, keepdims=True)
            bvals.append(m)
            bidx.append(idx)
            work = jnp.where(cols == idx, _NEG, work)
        bv = jnp.concatenate(bvals, axis=1)
        bi = jnp.concatenate(bidx, axis=1)
        u0 = jnp.max(jnp.where(cols == bi[:, 0:1], scores, _NEG),
                     axis=1, keepdims=True)

        @pl.when(j == 0)
        def _init():
            vals_s[...] = bv
            idx_s[...] = bi
            u_s[...] = u0

        @pl.when(j > 0)
        def _merge():
            pv = vals_s[...]
            pi = idx_s[...]
            better = bv[:, 0:1] > pv[:, 0:1]
            u_s[...] = jnp.where(better, u0, u_s[...])
            cv = jnp.concatenate([pv, bv], axis=1)
            ci = jnp.concatenate([pi, bi], axis=1)
            nvals = []
            nidx = []
            for _ in range(_TOPK):
                m = jnp.max(cv, axis=1, keepdims=True)
                sel = jnp.min(jnp.where(cv == m, ci, _M), axis=1, keepdims=True)
                nvals.append(m)
                nidx.append(sel)
                cv = jnp.where(ci == sel, _NEG, cv)
            vals_s[...] = jnp.concatenate(nvals, axis=1)
            idx_s[...] = jnp.concatenate(nidx, axis=1)

        @pl.when(j == _NBLK - 1)
        def _finalize():
            win_ref[...] = idx_s[...][:, 0:1]
            ms = u_s[...]
            gg = jax.nn.sigmoid(_BETA * (ms - _THETA))
            ms_ref[...] = ms
            g_ref[...] = gg
            y_ref[...] = ms * gg

    @pl.when(phase == 1)
    def _emit():
        tv = vals_s[...]
        ti = idx_s[...]
        acc = jnp.full((_B, _BLK), _NEG, dtype=jnp.float32)
        for k in range(_TOPK):
            acc = jnp.where(cols == ti[:, k:k + 1], tv[:, k:k + 1], acc)
        inh_ref[...] = acc


def kernel(x, keys, usage, s):
    xn = x / jnp.maximum(jnp.linalg.norm(x, axis=1, keepdims=True), 1e-12)
    usage2 = usage.reshape(1, _M)
    s2 = s.reshape(1, _M)

    grid = (2, _NBLK)
    out = pl.pallas_call(
        _fused_kernel,
        grid=grid,
        in_specs=[
            pl.BlockSpec((_B, _D), lambda p, j: (0, 0)),
            pl.BlockSpec((_BLK, _D), lambda p, j: (j, 0)),
            pl.BlockSpec((1, _BLK), lambda p, j: (0, j)),
            pl.BlockSpec((1, _BLK), lambda p, j: (0, j)),
        ],
        out_specs=[
            pl.BlockSpec((_B, _BLK),
                         lambda p, j: (0, jnp.where(p == 0, 0, j))),
            pl.BlockSpec((_B, 1), lambda p, j: (0, 0)),
            pl.BlockSpec((_B, 1), lambda p, j: (0, 0)),
            pl.BlockSpec((_B, 1), lambda p, j: (0, 0)),
            pl.BlockSpec((_B, 1), lambda p, j: (0, 0)),
        ],
        out_shape=[
            jax.ShapeDtypeStruct((_B, _M), jnp.float32),
            jax.ShapeDtypeStruct((_B, 1), jnp.int32),
            jax.ShapeDtypeStruct((_B, 1), jnp.float32),
            jax.ShapeDtypeStruct((_B, 1), jnp.float32),
            jax.ShapeDtypeStruct((_B, 1), jnp.float32),
        ],
        scratch_shapes=[
            pltpu.VMEM((_B, _TOPK), jnp.float32),
            pltpu.VMEM((_B, _TOPK), jnp.int32),
            pltpu.VMEM((_B, 1), jnp.float32),
        ],
    )(xn, keys, usage2, s2)
    inhibited, win, ms, y, g = out
    return (win[:, 0], ms[:, 0], y[:, 0], g[:, 0], inhibited)
